# Initial kernel scaffold; baseline (speedup 1.0000x reference)
import functools

import jax
import jax.numpy as jnp
from jax import lax
from jax.experimental import pallas as pl
from jax.experimental.pallas import tpu as pltpu
from jax.experimental.pallas import tpu_sc as plsc

N = 10000
E = 320000
D = 128
DEXT = 144
NC = 2
NS = 16
NW = NC * NS
CHUNK = 128
CPT = 79
NCHUNKS = NW * CPT
EPAD = NCHUNKS * CHUNK
NPAD = 10240
RPT = NPAD // NS
ZB = RPT // CHUNK

_mesh = plsc.VectorSubcoreMesh(
    core_axis_name="c", subcore_axis_name="s", num_cores=NC, num_subcores=NS)


def _make_agg(dext):
  def body(y_hbm, srcs_hbm, dsts_hbm, out_hbm, idx_s, idx_d, rows, acc, sem):
    cid = lax.axis_index("c")
    sid = lax.axis_index("s")
    wid = cid * NS + sid

    pltpu.sync_copy(srcs_hbm.at[pl.ds(wid * CPT, CPT)], idx_s)
    pltpu.sync_copy(dsts_hbm.at[pl.ds(wid * CPT, CPT)], idx_d)

    zeros16 = jnp.zeros((16,), jnp.float32)

    def zrow(i, carry):
      for j in range(dext // 16):
        rows[i, pl.ds(j * 16, 16)] = zeros16
      return carry

    lax.fori_loop(0, CHUNK, zrow, 0)
    for i in range(ZB):
      pltpu.sync_copy(rows, acc.at[pl.ds(sid * RPT + i * CHUNK, CHUNK)])
    plsc.subcore_barrier()

    def step(ci, carry):
      pltpu.async_copy(y_hbm.at[idx_s.at[ci]], rows, sem).wait()
      pltpu.sync_copy(rows, acc.at[idx_d.at[ci]], add=True)
      return carry

    lax.fori_loop(0, CPT, step, 0)
    plsc.subcore_barrier()

    for i in range(ZB):
      r0 = sid * RPT + i * CHUNK
      pltpu.sync_copy(acc.at[pl.ds(r0, CHUNK)], out_hbm.at[cid, pl.ds(r0, CHUNK)])

  return pl.kernel(
      body,
      out_type=jax.ShapeDtypeStruct((NC, NPAD, dext), jnp.float32),
      mesh=_mesh,
      scratch_types=[
          pltpu.VMEM((CPT, CHUNK), jnp.int32),
          pltpu.VMEM((CPT, CHUNK), jnp.int32),
          pltpu.VMEM((CHUNK, dext), jnp.float32),
          pltpu.VMEM_SHARED((NPAD, dext), jnp.float32),
          pltpu.SemaphoreType.DMA,
      ],
  )


_agg_ext = _make_agg(DEXT)
_agg = _make_agg(D)

_GRID = 10
_BR = N // _GRID


def _k1_body(x_ref, wl_ref, wr_ref, b_ref, yext_ref, z_ref):
  xb = x_ref[...]
  y = jnp.dot(xb, wl_ref[...], preferred_element_type=jnp.float32)
  ones_col = jnp.where(lax.broadcasted_iota(jnp.int32, (_BR, DEXT - D), 1) == 0,
                       1.0, 0.0)
  yext_ref[...] = jnp.concatenate([y, ones_col], axis=1)
  z_ref[...] = jnp.dot(xb, wr_ref[...], preferred_element_type=jnp.float32) + b_ref[...]


def _k1(x, wl, wr, b):
  return pl.pallas_call(
      _k1_body,
      grid=(_GRID,),
      in_specs=[
          pl.BlockSpec((_BR, D), lambda i: (i, 0)),
          pl.BlockSpec((D, D), lambda i: (0, 0)),
          pl.BlockSpec((D, D), lambda i: (0, 0)),
          pl.BlockSpec((1, D), lambda i: (0, 0)),
      ],
      out_specs=[
          pl.BlockSpec((_BR, DEXT), lambda i: (i, 0)),
          pl.BlockSpec((_BR, D), lambda i: (i, 0)),
      ],
      out_shape=[
          jax.ShapeDtypeStruct((N, DEXT), jnp.float32),
          jax.ShapeDtypeStruct((N, D), jnp.float32),
      ],
  )(x, wl, wr, b)


def _k2_body(s_ref, z1_ref, wl_ref, wr_ref, b_ref, y2_ref, z2_ref, ci_ref):
  s = s_ref[0] + s_ref[1]
  cnt = s[:, D:D + 1]
  ci = 1.0 / jnp.maximum(cnt, 1.0)
  h = jax.nn.relu(s[:, :D] * ci + z1_ref[...])
  y2_ref[...] = jnp.dot(h, wl_ref[...], preferred_element_type=jnp.float32)
  z2_ref[...] = jnp.dot(h, wr_ref[...], preferred_element_type=jnp.float32) + b_ref[...]
  ci_ref[...] = jnp.broadcast_to(ci, (_BR, D))


def _k2(s1, z1, wl, wr, b):
  return pl.pallas_call(
      _k2_body,
      grid=(_GRID,),
      in_specs=[
          pl.BlockSpec((NC, _BR, DEXT), lambda i: (0, i, 0)),
          pl.BlockSpec((_BR, D), lambda i: (i, 0)),
          pl.BlockSpec((D, D), lambda i: (0, 0)),
          pl.BlockSpec((D, D), lambda i: (0, 0)),
          pl.BlockSpec((1, D), lambda i: (0, 0)),
      ],
      out_specs=[
          pl.BlockSpec((_BR, D), lambda i: (i, 0)),
          pl.BlockSpec((_BR, D), lambda i: (i, 0)),
          pl.BlockSpec((_BR, D), lambda i: (i, 0)),
      ],
      out_shape=[
          jax.ShapeDtypeStruct((N, D), jnp.float32),
          jax.ShapeDtypeStruct((N, D), jnp.float32),
          jax.ShapeDtypeStruct((N, D), jnp.float32),
      ],
  )(s1, z1, wl, wr, b)


def _k3_body(s_ref, z2_ref, ci_ref, out_ref):
  out_ref[...] = (s_ref[0] + s_ref[1]) * ci_ref[...] + z2_ref[...]


def _k3(s2, z2, cinv):
  return pl.pallas_call(
      _k3_body,
      grid=(_GRID,),
      in_specs=[
          pl.BlockSpec((NC, _BR, D), lambda i: (0, i, 0)),
          pl.BlockSpec((_BR, D), lambda i: (i, 0)),
          pl.BlockSpec((_BR, D), lambda i: (i, 0)),
      ],
      out_specs=pl.BlockSpec((_BR, D), lambda i: (i, 0)),
      out_shape=jax.ShapeDtypeStruct((N, D), jnp.float32),
  )(s2, z2, cinv)


def kernel(x, edge_index, W1_l, b1_l, W1_r, W2_l, b2_l, W2_r):
  src = edge_index[0].astype(jnp.int32)
  dst = edge_index[1].astype(jnp.int32)
  srcs = jnp.concatenate(
      [src, jnp.zeros((EPAD - E,), jnp.int32)]).reshape(NCHUNKS, CHUNK)
  dsts = jnp.concatenate(
      [dst, jnp.full((EPAD - E,), N, jnp.int32)]).reshape(NCHUNKS, CHUNK)

  yext, z1 = _k1(x, W1_l, W1_r, b1_l.reshape(1, D))
  s1 = _agg_ext(yext, srcs, dsts)
  y2, z2, cinv = _k2(s1, z1, W2_l, W2_r, b2_l.reshape(1, D))
  s2 = _agg(y2, srcs, dsts)
  return _k3(s2, z2, cinv)


# speedup vs baseline: 3.4619x; 3.4619x over previous
import jax
import jax.numpy as jnp
from jax import lax
from jax.experimental import pallas as pl
from jax.experimental.pallas import tpu as pltpu
from jax.experimental.pallas import tpu_sc as plsc

N = 10000
E = 320000
D = 128
NC = 2
NS = 16
NW = NC * NS
CHUNK = 128
CPT = 80
NCHUNKS = NW * CPT
EPAD = NCHUNKS * CHUNK
NPAD = 10240
RPT = NPAD // NS
ZB = RPT // CHUNK

_mesh = plsc.VectorSubcoreMesh(
    core_axis_name="c", subcore_axis_name="s", num_cores=NC, num_subcores=NS)


def _make_agg(with_cnt):
  def body(y_hbm, srcs_hbm, dsts_hbm, *refs):
    if with_cnt:
      (out_hbm, cnt_hbm, idx_s, idx_d, rows, ones_v, zeros_v, acc, cacc,
       sem) = refs
    else:
      out_hbm, idx_s, idx_d, rows, acc, sem = refs
    cid = lax.axis_index("c")
    sid = lax.axis_index("s")
    wid = cid * NS + sid

    pltpu.sync_copy(srcs_hbm.at[pl.ds(wid * CPT, CPT)], idx_s)
    pltpu.sync_copy(dsts_hbm.at[pl.ds(wid * CPT, CPT)], idx_d)

    zeros16 = jnp.zeros((16,), jnp.float32)

    def zrow(i, carry):
      for j in range(D // 16):
        rows[i, pl.ds(j * 16, 16)] = zeros16
      return carry

    lax.fori_loop(0, CHUNK, zrow, 0)
    if with_cnt:
      for j in range(CHUNK // 16):
        ones_v[pl.ds(j * 16, 16)] = jnp.full((16,), 1.0, jnp.float32)
        zeros_v[pl.ds(j * 16, 16)] = zeros16
    for i in range(ZB):
      pltpu.sync_copy(rows, acc.at[pl.ds(sid * RPT + i * CHUNK, CHUNK)])
      if with_cnt:
        pltpu.sync_copy(zeros_v, cacc.at[pl.ds(sid * RPT + i * CHUNK, CHUNK)])
    plsc.subcore_barrier()

    def step(ci, carry):
      pltpu.async_copy(y_hbm.at[idx_s.at[ci]], rows, sem).wait()
      pltpu.sync_copy(rows, acc.at[idx_d.at[ci]], add=True)
      if with_cnt:
        pltpu.sync_copy(ones_v, cacc.at[idx_d.at[ci]], add=True)
      return carry

    lax.fori_loop(0, CPT, step, 0)
    plsc.subcore_barrier()

    for i in range(ZB):
      r0 = sid * RPT + i * CHUNK
      pltpu.sync_copy(acc.at[pl.ds(r0, CHUNK)], out_hbm.at[cid, pl.ds(r0, CHUNK)])
    if with_cnt:
      pltpu.sync_copy(cacc.at[pl.ds(sid * RPT, RPT)],
                      cnt_hbm.at[cid, pl.ds(sid * RPT, RPT)])

  out_type = [jax.ShapeDtypeStruct((NC, NPAD, D), jnp.float32)]
  scratch = [
      pltpu.VMEM((CPT, CHUNK), jnp.int32),
      pltpu.VMEM((CPT, CHUNK), jnp.int32),
      pltpu.VMEM((CHUNK, D), jnp.float32),
  ]
  if with_cnt:
    out_type.append(jax.ShapeDtypeStruct((NC, NPAD), jnp.float32))
    scratch += [
        pltpu.VMEM((CHUNK,), jnp.float32),
        pltpu.VMEM((CHUNK,), jnp.float32),
    ]
  scratch.append(pltpu.VMEM_SHARED((NPAD, D), jnp.float32))
  if with_cnt:
    scratch.append(pltpu.VMEM_SHARED((NPAD,), jnp.float32))
  scratch.append(pltpu.SemaphoreType.DMA)

  return pl.kernel(body, out_type=out_type, mesh=_mesh, scratch_types=scratch)


_agg_cnt = _make_agg(True)
_agg = _make_agg(False)

_GRID = 10
_BR = 1024


def _k1_body(x_ref, wl_ref, wr_ref, b_ref, y_ref, z_ref):
  xb = x_ref[...]
  y_ref[...] = jnp.dot(xb, wl_ref[...], preferred_element_type=jnp.float32)
  z_ref[...] = jnp.dot(xb, wr_ref[...], preferred_element_type=jnp.float32) + b_ref[...]


def _k1(x, wl, wr, b):
  return pl.pallas_call(
      _k1_body,
      grid=(_GRID,),
      in_specs=[
          pl.BlockSpec((_BR, D), lambda i: (i, 0)),
          pl.BlockSpec((D, D), lambda i: (0, 0)),
          pl.BlockSpec((D, D), lambda i: (0, 0)),
          pl.BlockSpec((1, D), lambda i: (0, 0)),
      ],
      out_specs=[
          pl.BlockSpec((_BR, D), lambda i: (i, 0)),
          pl.BlockSpec((_BR, D), lambda i: (i, 0)),
      ],
      out_shape=[
          jax.ShapeDtypeStruct((N, D), jnp.float32),
          jax.ShapeDtypeStruct((N, D), jnp.float32),
      ],
  )(x, wl, wr, b)


def _k2_body(s_ref, cnt_ref, z1_ref, wl_ref, wr_ref, b_ref,
             y2_ref, z2_ref, ci_ref):
  cnt = cnt_ref[0:1, :] + cnt_ref[1:2, :]
  ci = jnp.transpose(1.0 / jnp.maximum(cnt, 1.0), (1, 0))
  s = s_ref[0] + s_ref[1]
  h = jax.nn.relu(s * ci + z1_ref[...])
  y2_ref[...] = jnp.dot(h, wl_ref[...], preferred_element_type=jnp.float32)
  z2_ref[...] = jnp.dot(h, wr_ref[...], preferred_element_type=jnp.float32) + b_ref[...]
  ci_ref[...] = jnp.broadcast_to(ci, (_BR, D))


def _k2(s1, cnt, z1, wl, wr, b):
  return pl.pallas_call(
      _k2_body,
      grid=(_GRID,),
      in_specs=[
          pl.BlockSpec((NC, _BR, D), lambda i: (0, i, 0)),
          pl.BlockSpec((NC, _BR), lambda i: (0, i)),
          pl.BlockSpec((_BR, D), lambda i: (i, 0)),
          pl.BlockSpec((D, D), lambda i: (0, 0)),
          pl.BlockSpec((D, D), lambda i: (0, 0)),
          pl.BlockSpec((1, D), lambda i: (0, 0)),
      ],
      out_specs=[
          pl.BlockSpec((_BR, D), lambda i: (i, 0)),
          pl.BlockSpec((_BR, D), lambda i: (i, 0)),
          pl.BlockSpec((_BR, D), lambda i: (i, 0)),
      ],
      out_shape=[
          jax.ShapeDtypeStruct((N, D), jnp.float32),
          jax.ShapeDtypeStruct((N, D), jnp.float32),
          jax.ShapeDtypeStruct((N, D), jnp.float32),
      ],
  )(s1, cnt, z1, wl, wr, b)


def _k3_body(s_ref, z2_ref, ci_ref, out_ref):
  out_ref[...] = (s_ref[0] + s_ref[1]) * ci_ref[...] + z2_ref[...]


def _k3(s2, z2, cinv):
  return pl.pallas_call(
      _k3_body,
      grid=(_GRID,),
      in_specs=[
          pl.BlockSpec((NC, _BR, D), lambda i: (0, i, 0)),
          pl.BlockSpec((_BR, D), lambda i: (i, 0)),
          pl.BlockSpec((_BR, D), lambda i: (i, 0)),
      ],
      out_specs=pl.BlockSpec((_BR, D), lambda i: (i, 0)),
      out_shape=jax.ShapeDtypeStruct((N, D), jnp.float32),
  )(s2, z2, cinv)


def kernel(x, edge_index, W1_l, b1_l, W1_r, W2_l, b2_l, W2_r):
  src = edge_index[0].astype(jnp.int32)
  dst = edge_index[1].astype(jnp.int32)
  srcs = jnp.concatenate(
      [src, jnp.zeros((EPAD - E,), jnp.int32)]).reshape(NCHUNKS, CHUNK)
  dsts = jnp.concatenate(
      [dst, jnp.full((EPAD - E,), N, jnp.int32)]).reshape(NCHUNKS, CHUNK)

  y1, z1 = _k1(x, W1_l, W1_r, b1_l.reshape(1, D))
  s1, cnt = _agg_cnt(y1, srcs, dsts)
  y2, z2, cinv = _k2(s1, cnt, z1, W2_l, W2_r, b2_l.reshape(1, D))
  (s2,) = _agg(y2, srcs, dsts)
  return _k3(s2, z2, cinv)


# speedup vs baseline: 6.5513x; 1.8924x over previous
import jax
import jax.numpy as jnp
from jax import lax
from jax.experimental import pallas as pl
from jax.experimental.pallas import tpu as pltpu
from jax.experimental.pallas import tpu_sc as plsc

N = 10000
E = 320000
D = 128
NC = 2
NS = 16
NW = NC * NS
CHUNK = 128
CPT = 80
NCHUNKS = NW * CPT
EPAD = NCHUNKS * CHUNK
NPAD = 10112
RPT = NPAD // NS
HALF = CPT // 2
CNPAD = 10240
CRPT = CNPAD // NS

_mesh = plsc.VectorSubcoreMesh(
    core_axis_name="c", subcore_axis_name="s", num_cores=NC, num_subcores=NS)


def _make_agg(with_cnt):
  def body(y_hbm, srcs_hbm, dsts_hbm, *refs):
    if with_cnt:
      (out_hbm, cnt0_hbm, cnt1_hbm, idx_s, idx_d, rows0, rows1, ones_v, acc,
       cacc, sem0, sem1) = refs
    else:
      out_hbm, idx_s, idx_d, rows0, rows1, acc, sem0, sem1 = refs
    cid = lax.axis_index("c")
    sid = lax.axis_index("s")
    wid = cid * NS + sid

    pltpu.sync_copy(srcs_hbm.at[pl.ds(wid * CPT, CPT)], idx_s.at[pl.ds(0, CPT)])
    pltpu.sync_copy(dsts_hbm.at[pl.ds(wid * CPT, HALF)], idx_d)
    izeros16 = jnp.zeros((16,), jnp.int32)
    for j in range(CHUNK // 16):
      idx_s[CPT, pl.ds(j * 16, 16)] = izeros16

    zeros16 = jnp.zeros((16,), jnp.float32)

    def zrow(i, carry):
      for j in range(D // 16):
        rows0[i, pl.ds(j * 16, 16)] = zeros16
      return carry

    lax.fori_loop(0, CHUNK, zrow, 0)
    if with_cnt:
      for j in range(CHUNK // 16):
        ones_v[pl.ds(j * 16, 16)] = jnp.full((16,), 1.0, jnp.float32)
    for i in range(4):
      pltpu.sync_copy(rows0, acc.at[pl.ds(sid * RPT + i * CHUNK, CHUNK)])
    pltpu.sync_copy(rows0, acc.at[pl.ds(sid * RPT + RPT - CHUNK, CHUNK)])
    if with_cnt:
      for i in range(CRPT // CHUNK):
        pltpu.sync_copy(rows0.at[0], cacc.at[pl.ds(sid * CRPT + i * CHUNK, CHUNK)])
    plsc.subcore_barrier()

    pltpu.async_copy(y_hbm.at[idx_s.at[0]], rows0, sem0)

    def make_step(doff):
      def step2(p, carry):
        c0 = 2 * p
        pltpu.async_copy(y_hbm.at[idx_s.at[c0 + 1]], rows1, sem1)
        pltpu.make_async_copy(y_hbm.at[idx_s.at[c0]], rows0, sem0).wait()
        pltpu.sync_copy(rows0, acc.at[idx_d.at[c0 - doff]], add=True)
        if with_cnt:
          pltpu.sync_copy(ones_v, cacc.at[idx_d.at[c0 - doff]], add=True)
        pltpu.async_copy(y_hbm.at[idx_s.at[c0 + 2]], rows0, sem0)
        pltpu.make_async_copy(y_hbm.at[idx_s.at[c0 + 1]], rows1, sem1).wait()
        pltpu.sync_copy(rows1, acc.at[idx_d.at[c0 + 1 - doff]], add=True)
        if with_cnt:
          pltpu.sync_copy(ones_v, cacc.at[idx_d.at[c0 + 1 - doff]], add=True)
        return carry
      return step2

    lax.fori_loop(0, HALF // 2, make_step(0), 0)
    pltpu.sync_copy(dsts_hbm.at[pl.ds(wid * CPT + HALF, HALF)], idx_d)
    lax.fori_loop(HALF // 2, CPT // 2, make_step(HALF), 0)
    pltpu.make_async_copy(y_hbm.at[idx_s.at[CPT]], rows0, sem0).wait()
    plsc.subcore_barrier()

    pltpu.sync_copy(acc.at[pl.ds(sid * RPT, RPT)],
                    out_hbm.at[cid, pl.ds(sid * RPT, RPT)])
    if with_cnt:
      @pl.when(cid == 0)
      def _():
        pltpu.sync_copy(cacc.at[pl.ds(sid * CRPT, CRPT)],
                        cnt0_hbm.at[pl.ds(sid * CRPT, CRPT)])

      @pl.when(cid == 1)
      def _():
        pltpu.sync_copy(cacc.at[pl.ds(sid * CRPT, CRPT)],
                        cnt1_hbm.at[pl.ds(sid * CRPT, CRPT)])

  out_type = [jax.ShapeDtypeStruct((NC, NPAD, D), jnp.float32)]
  scratch = [
      pltpu.VMEM((CPT + 1, CHUNK), jnp.int32),
      pltpu.VMEM((HALF, CHUNK), jnp.int32),
      pltpu.VMEM((CHUNK, D), jnp.float32),
      pltpu.VMEM((CHUNK, D), jnp.float32),
  ]
  if with_cnt:
    out_type.append(jax.ShapeDtypeStruct((CNPAD,), jnp.float32))
    out_type.append(jax.ShapeDtypeStruct((CNPAD,), jnp.float32))
    scratch += [
        pltpu.VMEM((CHUNK,), jnp.float32),
    ]
  scratch.append(pltpu.VMEM_SHARED((NPAD, D), jnp.float32))
  if with_cnt:
    scratch.append(pltpu.VMEM_SHARED((CNPAD,), jnp.float32))
  scratch += [pltpu.SemaphoreType.DMA, pltpu.SemaphoreType.DMA]

  return pl.kernel(body, out_type=out_type, mesh=_mesh, scratch_types=scratch)


_agg_cnt = _make_agg(True)
_agg = _make_agg(False)

_GRID = 10
_BR = 1024


def _k1_body(x_ref, wl_ref, wr_ref, b_ref, y_ref, z_ref):
  xb = x_ref[...]
  y_ref[...] = jnp.dot(xb, wl_ref[...], preferred_element_type=jnp.float32)
  z_ref[...] = jnp.dot(xb, wr_ref[...], preferred_element_type=jnp.float32) + b_ref[...]


def _k1(x, wl, wr, b):
  return pl.pallas_call(
      _k1_body,
      grid=(_GRID,),
      in_specs=[
          pl.BlockSpec((_BR, D), lambda i: (i, 0)),
          pl.BlockSpec((D, D), lambda i: (0, 0)),
          pl.BlockSpec((D, D), lambda i: (0, 0)),
          pl.BlockSpec((1, D), lambda i: (0, 0)),
      ],
      out_specs=[
          pl.BlockSpec((_BR, D), lambda i: (i, 0)),
          pl.BlockSpec((_BR, D), lambda i: (i, 0)),
      ],
      out_shape=[
          jax.ShapeDtypeStruct((N, D), jnp.float32),
          jax.ShapeDtypeStruct((N, D), jnp.float32),
      ],
  )(x, wl, wr, b)


def _k2_body(s_ref, cnt_ref, z1_ref, wl_ref, wr_ref, b_ref,
             y2_ref, z2_ref, ci_ref):
  cnt = cnt_ref[0:1, :] + cnt_ref[1:2, :]
  ci = jnp.transpose(1.0 / jnp.maximum(cnt, 1.0), (1, 0))
  s = s_ref[0] + s_ref[1]
  h = jax.nn.relu(s * ci + z1_ref[...])
  y2_ref[...] = jnp.dot(h, wl_ref[...], preferred_element_type=jnp.float32)
  z2_ref[...] = jnp.dot(h, wr_ref[...], preferred_element_type=jnp.float32) + b_ref[...]
  ci_ref[...] = jnp.broadcast_to(ci, (_BR, D))


def _k2(s1, cnt, z1, wl, wr, b):
  return pl.pallas_call(
      _k2_body,
      grid=(_GRID,),
      in_specs=[
          pl.BlockSpec((NC, _BR, D), lambda i: (0, i, 0)),
          pl.BlockSpec((NC, _BR), lambda i: (0, i)),
          pl.BlockSpec((_BR, D), lambda i: (i, 0)),
          pl.BlockSpec((D, D), lambda i: (0, 0)),
          pl.BlockSpec((D, D), lambda i: (0, 0)),
          pl.BlockSpec((1, D), lambda i: (0, 0)),
      ],
      out_specs=[
          pl.BlockSpec((_BR, D), lambda i: (i, 0)),
          pl.BlockSpec((_BR, D), lambda i: (i, 0)),
          pl.BlockSpec((_BR, D), lambda i: (i, 0)),
      ],
      out_shape=[
          jax.ShapeDtypeStruct((N, D), jnp.float32),
          jax.ShapeDtypeStruct((N, D), jnp.float32),
          jax.ShapeDtypeStruct((N, D), jnp.float32),
      ],
  )(s1, cnt, z1, wl, wr, b)


def _k3_body(s_ref, z2_ref, ci_ref, out_ref):
  out_ref[...] = (s_ref[0] + s_ref[1]) * ci_ref[...] + z2_ref[...]


def _k3(s2, z2, cinv):
  return pl.pallas_call(
      _k3_body,
      grid=(_GRID,),
      in_specs=[
          pl.BlockSpec((NC, _BR, D), lambda i: (0, i, 0)),
          pl.BlockSpec((_BR, D), lambda i: (i, 0)),
          pl.BlockSpec((_BR, D), lambda i: (i, 0)),
      ],
      out_specs=pl.BlockSpec((_BR, D), lambda i: (i, 0)),
      out_shape=jax.ShapeDtypeStruct((N, D), jnp.float32),
  )(s2, z2, cinv)


def kernel(x, edge_index, W1_l, b1_l, W1_r, W2_l, b2_l, W2_r):
  src = edge_index[0].astype(jnp.int32)
  dst = edge_index[1].astype(jnp.int32)
  pad = jnp.arange(EPAD - E, dtype=jnp.int32)
  srcs = jnp.concatenate([src, pad % N]).reshape(NCHUNKS, CHUNK)
  dsts = jnp.concatenate([dst, N + pad % (NPAD - N)]).reshape(NCHUNKS, CHUNK)

  y1, z1 = _k1(x, W1_l, W1_r, b1_l.reshape(1, D))
  s1, cnt0, cnt1 = _agg_cnt(y1, srcs, dsts)
  cnt = jnp.stack([cnt0, cnt1])
  y2, z2, cinv = _k2(s1, cnt, z1, W2_l, W2_r, b2_l.reshape(1, D))
  (s2,) = _agg(y2, srcs, dsts)
  return _k3(s2, z2, cinv)
